# Initial kernel scaffold; baseline (speedup 1.0000x reference)
import functools

import jax
import jax.numpy as jnp
import numpy as np
from jax import lax
from jax.experimental import pallas as pl
from jax.experimental.pallas import tpu as pltpu
from jax.experimental.pallas import tpu_sc as plsc

N_POINTS = 10000
N_VIEWS = 3
SCALES = (56, 28, 14, 7)
CHANNELS = (64, 128, 256, 512)
C_TOTAL = 960
OUT_COLS = 3 + 3 * C_TOTAL
CHUNK = 16
N_CHUNKS = N_POINTS // CHUNK
OUT_CHUNK = CHUNK * OUT_COLS

NUM_CORES = 2
NUM_SUBCORES = 16
NUM_WORKERS = NUM_CORES * NUM_SUBCORES
BASE_CHUNKS = N_CHUNKS // NUM_WORKERS
EXTRA = N_CHUNKS - BASE_CHUNKS * NUM_WORKERS

CH_OFF = (0, 64, 192, 448)


def _normal(v):
    return v / jnp.linalg.norm(v)


def _camera_mat(param):
    theta = param[0] * np.pi / 180.0
    camy = param[3] * jnp.sin(param[1] * np.pi / 180.0)
    lens = param[3] * jnp.cos(param[1] * np.pi / 180.0)
    camx = lens * jnp.cos(theta)
    camz = lens * jnp.sin(theta)
    Z = jnp.stack([camx, camy, camz])
    x = camy * jnp.cos(theta + np.pi)
    z = camy * jnp.sin(theta + np.pi)
    Y = jnp.stack([x, lens, z])
    X = jnp.cross(Y, Z)
    cm_mat = jnp.stack([_normal(X), _normal(Y), _normal(Z)])
    return cm_mat, Z


def _camera_trans(param, xyz):
    c, o = _camera_mat(param)
    return (xyz - o) @ c.T


def _camera_trans_inv(param, xyz):
    c, o = _camera_mat(param)
    return xyz @ jnp.linalg.inv(c.T) + o


def _flat_indices(inputs, cameras):
    flat = [[] for _ in SCALES]
    for i in range(N_VIEWS):
        point_origin = _camera_trans_inv(cameras[0], inputs)
        point_current = _camera_trans(cameras[i], point_origin)
        X = point_current[:, 0]
        Y = point_current[:, 1]
        Z = point_current[:, 2]
        h = 248.0 * ((-Y) / (-Z)) + 112.0
        w = 248.0 * (X / (-Z)) + 112.0
        h = jnp.minimum(jnp.maximum(h, 0.0), 223.0)
        w = jnp.minimum(jnp.maximum(w, 0.0), 223.0)
        n = jnp.full(h.shape, float(i), dtype=jnp.float32)
        indeces = jnp.stack([n, h, w], 1)
        for j, s in enumerate(SCALES):
            idx = (indeces / (224.0 / float(s))).astype(jnp.int32)
            flat[j].append((idx[:, 0] * s + idx[:, 1]) * s + idx[:, 2])
    packed = []
    for j in range(len(SCALES)):
        a = jnp.stack(flat[j], 0)
        a = a.reshape(N_VIEWS, N_CHUNKS, CHUNK).transpose(1, 0, 2)
        packed.append(a.reshape(N_CHUNKS, N_VIEWS * CHUNK))
    return jnp.stack(packed, 1)


def _sc_body(t0, t1, t2, t3, idxp, coords, out,
             idxv, coordv, r0, r1, r2, r3, outbuf,
             s0, s1, s2, s3):
    tables = (t0, t1, t2, t3)
    rows = (r0, r1, r2, r3)
    sems = (s0, s1, s2, s3)
    wid = lax.axis_index("s") * NUM_CORES + lax.axis_index("c")
    start = wid * BASE_CHUNKS + lax.min(wid, EXTRA)
    count = BASE_CHUNKS + jnp.where(wid < EXTRA, 1, 0)

    lanes = lax.broadcasted_iota(jnp.int32, (CHUNK,), 0)
    rowbase = lanes * OUT_COLS

    third = jnp.float32(1.0 / 3.0)

    def chunk_body(t, carry):
        c = start + t
        pltpu.sync_copy(idxp.at[c], idxv)
        pltpu.sync_copy(coords.at[c], coordv)
        cps = [pltpu.async_copy(tables[j].at[idxv.at[j]], rows[j], sems[j])
               for j in range(4)]
        for cp in cps:
            cp.wait()
        for d in range(3):
            plsc.store_scatter(outbuf, [rowbase + d], coordv[d, :])

        def point_body(p, carry2):
            obase = p * OUT_COLS + 3
            for j in range(4):
                cj = CHANNELS[j]
                for cb in range(cj // 16):
                    off = cb * 16
                    x0 = rows[j][p, pl.ds(off, 16)]
                    x1 = rows[j][CHUNK + p, pl.ds(off, 16)]
                    x2 = rows[j][2 * CHUNK + p, pl.ds(off, 16)]
                    mx = lax.max(lax.max(x0, x1), x2)
                    mn = (x0 + x1 + x2) * third
                    d0 = x0 - mn
                    d1 = x1 - mn
                    d2 = x2 - mn
                    var = (d0 * d0 + d1 * d1 + d2 * d2) * third
                    varc = lax.max(var, jnp.float32(1e-35))
                    iy = jnp.int32(0x5F3759DF) - lax.shift_right_logical(
                        plsc.bitcast(varc, jnp.int32), 1)
                    y = plsc.bitcast(iy, jnp.float32)
                    hv = varc * jnp.float32(0.5)
                    y = y * (jnp.float32(1.5) - hv * y * y)
                    y = y * (jnp.float32(1.5) - hv * y * y)
                    y = y * (jnp.float32(1.5) - hv * y * y)
                    sd = varc * y
                    col = obase + CH_OFF[j] + off
                    outbuf[pl.ds(col, 16)] = mx
                    outbuf[pl.ds(col + C_TOTAL, 16)] = mn
                    outbuf[pl.ds(col + 2 * C_TOTAL, 16)] = sd
            return carry2

        lax.fori_loop(0, CHUNK, point_body, 0)
        pltpu.sync_copy(outbuf, out.at[pl.ds(c * OUT_CHUNK, OUT_CHUNK)])
        return carry

    lax.fori_loop(0, count, chunk_body, 0)


def kernel(inputs, img_feat_0, img_feat_1, img_feat_2, img_feat_3, cameras):
    idxp = _flat_indices(inputs, cameras)
    coords = inputs.reshape(N_CHUNKS, CHUNK, 3).transpose(0, 2, 1)
    feats = (img_feat_0, img_feat_1, img_feat_2, img_feat_3)
    tables = [f.reshape(N_VIEWS * s * s, c)
              for f, s, c in zip(feats, SCALES, CHANNELS)]

    mesh = plsc.VectorSubcoreMesh(core_axis_name="c", subcore_axis_name="s")
    run = functools.partial(
        pl.kernel,
        out_type=jax.ShapeDtypeStruct((N_POINTS * OUT_COLS,), jnp.float32),
        mesh=mesh,
        scratch_types=[
            pltpu.VMEM((4, N_VIEWS * CHUNK), jnp.int32),
            pltpu.VMEM((3, CHUNK), jnp.float32),
            pltpu.VMEM((N_VIEWS * CHUNK, CHANNELS[0]), jnp.float32),
            pltpu.VMEM((N_VIEWS * CHUNK, CHANNELS[1]), jnp.float32),
            pltpu.VMEM((N_VIEWS * CHUNK, CHANNELS[2]), jnp.float32),
            pltpu.VMEM((N_VIEWS * CHUNK, CHANNELS[3]), jnp.float32),
            pltpu.VMEM((OUT_CHUNK,), jnp.float32),
            pltpu.SemaphoreType.DMA,
            pltpu.SemaphoreType.DMA,
            pltpu.SemaphoreType.DMA,
            pltpu.SemaphoreType.DMA,
        ],
    )(_sc_body)
    out = run(tables[0], tables[1], tables[2], tables[3], idxp, coords)
    return out.reshape(N_POINTS, OUT_COLS)


# speedup vs baseline: 1.2798x; 1.2798x over previous
import functools

import jax
import jax.numpy as jnp
import numpy as np
from jax import lax
from jax.experimental import pallas as pl
from jax.experimental.pallas import tpu as pltpu
from jax.experimental.pallas import tpu_sc as plsc

N_POINTS = 10000
N_VIEWS = 3
SCALES = (56, 28, 14, 7)
CHANNELS = (64, 128, 256, 512)
C_TOTAL = 960
OUT_COLS = 3 + 3 * C_TOTAL
CHUNK = 16
N_CHUNKS = N_POINTS // CHUNK
OUT_CHUNK = CHUNK * OUT_COLS

NUM_CORES = 2
NUM_SUBCORES = 16
NUM_WORKERS = NUM_CORES * NUM_SUBCORES
BASE_CHUNKS = N_CHUNKS // NUM_WORKERS
EXTRA = N_CHUNKS - BASE_CHUNKS * NUM_WORKERS

CH_OFF = (0, 64, 192, 448)


def _normal(v):
    return v / jnp.linalg.norm(v)


def _camera_mat(param):
    theta = param[0] * np.pi / 180.0
    camy = param[3] * jnp.sin(param[1] * np.pi / 180.0)
    lens = param[3] * jnp.cos(param[1] * np.pi / 180.0)
    camx = lens * jnp.cos(theta)
    camz = lens * jnp.sin(theta)
    Z = jnp.stack([camx, camy, camz])
    x = camy * jnp.cos(theta + np.pi)
    z = camy * jnp.sin(theta + np.pi)
    Y = jnp.stack([x, lens, z])
    X = jnp.cross(Y, Z)
    cm_mat = jnp.stack([_normal(X), _normal(Y), _normal(Z)])
    return cm_mat, Z


def _camera_trans(param, xyz):
    c, o = _camera_mat(param)
    return (xyz - o) @ c.T


def _camera_trans_inv(param, xyz):
    c, o = _camera_mat(param)
    return xyz @ jnp.linalg.inv(c.T) + o


def _flat_indices(inputs, cameras):
    flat = [[] for _ in SCALES]
    for i in range(N_VIEWS):
        point_origin = _camera_trans_inv(cameras[0], inputs)
        point_current = _camera_trans(cameras[i], point_origin)
        X = point_current[:, 0]
        Y = point_current[:, 1]
        Z = point_current[:, 2]
        h = 248.0 * ((-Y) / (-Z)) + 112.0
        w = 248.0 * (X / (-Z)) + 112.0
        h = jnp.minimum(jnp.maximum(h, 0.0), 223.0)
        w = jnp.minimum(jnp.maximum(w, 0.0), 223.0)
        n = jnp.full(h.shape, float(i), dtype=jnp.float32)
        indeces = jnp.stack([n, h, w], 1)
        for j, s in enumerate(SCALES):
            idx = (indeces / (224.0 / float(s))).astype(jnp.int32)
            flat[j].append((idx[:, 0] * s + idx[:, 1]) * s + idx[:, 2])
    packed = []
    for j in range(len(SCALES)):
        a = jnp.stack(flat[j], 0)
        a = a.reshape(N_VIEWS, N_CHUNKS, CHUNK).transpose(1, 0, 2)
        packed.append(a.reshape(N_CHUNKS, N_VIEWS * CHUNK))
    return jnp.stack(packed, 1)


def _sc_body(t0, t1, t2, t3, idxp, coords, out,
             idxv, coordv, r0, r1, r2, r3, outbuf,
             s0, s1, s2, s3):
    tables = (t0, t1, t2, t3)
    rows = (r0, r1, r2, r3)
    sems = (s0, s1, s2, s3)
    wid = lax.axis_index("s") * NUM_CORES + lax.axis_index("c")
    start = wid * BASE_CHUNKS + lax.min(wid, EXTRA)
    count = BASE_CHUNKS + jnp.where(wid < EXTRA, 1, 0)

    third = jnp.float32(1.0 / 3.0)

    def chunk_body(t, carry):
        c = start + t
        pltpu.sync_copy(idxp.at[c], idxv)
        pltpu.sync_copy(coords.at[c], coordv)
        cps = [pltpu.async_copy(tables[j].at[idxv.at[j]], rows[j], sems[j])
               for j in range(4)]
        for cp in cps:
            cp.wait()
        def point_body(p, carry2):
            outbuf[pl.ds(p * OUT_COLS, 16)] = coordv[p, :]
            obase = p * OUT_COLS + 3
            for j in range(4):
                cj = CHANNELS[j]
                for cb in range(cj // 16):
                    off = cb * 16
                    x0 = rows[j][p, pl.ds(off, 16)]
                    x1 = rows[j][CHUNK + p, pl.ds(off, 16)]
                    x2 = rows[j][2 * CHUNK + p, pl.ds(off, 16)]
                    mx = lax.max(lax.max(x0, x1), x2)
                    mn = (x0 + x1 + x2) * third
                    d0 = x0 - mn
                    d1 = x1 - mn
                    d2 = x2 - mn
                    var = (d0 * d0 + d1 * d1 + d2 * d2) * third
                    varc = lax.max(var, jnp.float32(1e-35))
                    iy = jnp.int32(0x5F3759DF) - lax.shift_right_logical(
                        lax.bitcast_convert_type(varc, jnp.int32), 1)
                    y = lax.bitcast_convert_type(iy, jnp.float32)
                    hv = varc * jnp.float32(0.5)
                    y = y * (jnp.float32(1.5) - hv * y * y)
                    y = y * (jnp.float32(1.5) - hv * y * y)
                    y = y * (jnp.float32(1.5) - hv * y * y)
                    sd = varc * y
                    col = obase + CH_OFF[j] + off
                    outbuf[pl.ds(col, 16)] = mx
                    outbuf[pl.ds(col + C_TOTAL, 16)] = mn
                    outbuf[pl.ds(col + 2 * C_TOTAL, 16)] = sd
            return carry2

        lax.fori_loop(0, CHUNK, point_body, 0)
        pltpu.sync_copy(outbuf, out.at[pl.ds(c * OUT_CHUNK, OUT_CHUNK)])
        return carry

    lax.fori_loop(0, count, chunk_body, 0)


def kernel(inputs, img_feat_0, img_feat_1, img_feat_2, img_feat_3, cameras):
    idxp = _flat_indices(inputs, cameras)
    coords = jnp.pad(inputs.reshape(N_CHUNKS, CHUNK, 3),
                     ((0, 0), (0, 0), (0, 13)))
    feats = (img_feat_0, img_feat_1, img_feat_2, img_feat_3)
    tables = [f.reshape(N_VIEWS * s * s, c)
              for f, s, c in zip(feats, SCALES, CHANNELS)]

    mesh = plsc.VectorSubcoreMesh(core_axis_name="c", subcore_axis_name="s")
    run = functools.partial(
        pl.kernel,
        out_type=jax.ShapeDtypeStruct((N_POINTS * OUT_COLS,), jnp.float32),
        mesh=mesh,
        compiler_params=pltpu.CompilerParams(use_tc_tiling_on_sc=False),
        scratch_types=[
            pltpu.VMEM((4, N_VIEWS * CHUNK), jnp.int32),
            pltpu.VMEM((CHUNK, 16), jnp.float32),
            pltpu.VMEM((N_VIEWS * CHUNK, CHANNELS[0]), jnp.float32),
            pltpu.VMEM((N_VIEWS * CHUNK, CHANNELS[1]), jnp.float32),
            pltpu.VMEM((N_VIEWS * CHUNK, CHANNELS[2]), jnp.float32),
            pltpu.VMEM((N_VIEWS * CHUNK, CHANNELS[3]), jnp.float32),
            pltpu.VMEM((OUT_CHUNK,), jnp.float32),
            pltpu.SemaphoreType.DMA,
            pltpu.SemaphoreType.DMA,
            pltpu.SemaphoreType.DMA,
            pltpu.SemaphoreType.DMA,
        ],
    )(_sc_body)
    out = run(tables[0], tables[1], tables[2], tables[3], idxp, coords)
    return out.reshape(N_POINTS, OUT_COLS)


# speedup vs baseline: 1.8763x; 1.4661x over previous
import functools

import jax
import jax.numpy as jnp
import numpy as np
from jax import lax
from jax.experimental import pallas as pl
from jax.experimental.pallas import tpu as pltpu
from jax.experimental.pallas import tpu_sc as plsc

N_POINTS = 10000
N_VIEWS = 3
SCALES = (56, 28, 14, 7)
CHANNELS = (64, 128, 256, 512)
C_TOTAL = 960
OUT_COLS = 3 + 3 * C_TOTAL
CHUNK = 16
N_CHUNKS = N_POINTS // CHUNK
OUT_CHUNK = CHUNK * OUT_COLS

NUM_CORES = 2
NUM_SUBCORES = 16
NUM_WORKERS = NUM_CORES * NUM_SUBCORES
BASE_CHUNKS = N_CHUNKS // NUM_WORKERS
EXTRA = N_CHUNKS - BASE_CHUNKS * NUM_WORKERS

CH_OFF = (0, 64, 192, 448)


def _normal(v):
    return v / jnp.linalg.norm(v)


def _camera_mat(param):
    theta = param[0] * np.pi / 180.0
    camy = param[3] * jnp.sin(param[1] * np.pi / 180.0)
    lens = param[3] * jnp.cos(param[1] * np.pi / 180.0)
    camx = lens * jnp.cos(theta)
    camz = lens * jnp.sin(theta)
    Z = jnp.stack([camx, camy, camz])
    x = camy * jnp.cos(theta + np.pi)
    z = camy * jnp.sin(theta + np.pi)
    Y = jnp.stack([x, lens, z])
    X = jnp.cross(Y, Z)
    cm_mat = jnp.stack([_normal(X), _normal(Y), _normal(Z)])
    return cm_mat, Z


def _camera_trans(param, xyz):
    c, o = _camera_mat(param)
    return (xyz - o) @ c.T


def _camera_trans_inv(param, xyz):
    c, o = _camera_mat(param)
    return xyz @ jnp.linalg.inv(c.T) + o


def _flat_indices(inputs, cameras):
    flat = [[] for _ in SCALES]
    for i in range(N_VIEWS):
        point_origin = _camera_trans_inv(cameras[0], inputs)
        point_current = _camera_trans(cameras[i], point_origin)
        X = point_current[:, 0]
        Y = point_current[:, 1]
        Z = point_current[:, 2]
        h = 248.0 * ((-Y) / (-Z)) + 112.0
        w = 248.0 * (X / (-Z)) + 112.0
        h = jnp.minimum(jnp.maximum(h, 0.0), 223.0)
        w = jnp.minimum(jnp.maximum(w, 0.0), 223.0)
        n = jnp.full(h.shape, float(i), dtype=jnp.float32)
        indeces = jnp.stack([n, h, w], 1)
        for j, s in enumerate(SCALES):
            idx = (indeces / (224.0 / float(s))).astype(jnp.int32)
            flat[j].append((idx[:, 0] * s + idx[:, 1]) * s + idx[:, 2])
    packed = []
    for j in range(len(SCALES)):
        a = jnp.stack(flat[j], 0)
        a = a.reshape(N_VIEWS, N_CHUNKS, CHUNK).transpose(1, 0, 2)
        packed.append(a.reshape(N_CHUNKS, N_VIEWS * CHUNK))
    return jnp.stack(packed, 1)


def _sc_body(t0, t1, t2, t3, idxp, coords, out,
             idxv, coordv, r0, r1, r2, r3, outbuf,
             s0, s1, s2, s3):
    tables = (t0, t1, t2, t3)
    rows = (r0, r1, r2, r3)
    sems = (s0, s1, s2, s3)
    wid = lax.axis_index("s") * NUM_CORES + lax.axis_index("c")
    start = wid * BASE_CHUNKS + lax.min(wid, EXTRA)
    count = BASE_CHUNKS + jnp.where(wid < EXTRA, 1, 0)

    third = jnp.float32(1.0 / 3.0)

    def chunk_body(t, carry):
        c = start + t
        pltpu.sync_copy(idxp.at[c], idxv)
        pltpu.sync_copy(coords.at[c], coordv)
        cps = [pltpu.async_copy(tables[j].at[idxv.at[j]], rows[j], sems[j])
               for j in range(4)]
        for cp in cps:
            cp.wait()
        @plsc.parallel_loop(0, CHUNK)
        def _coord_loop(p):
            outbuf[pl.ds(p * OUT_COLS, 16)] = coordv[p, :]

        for j in range(4):
            cj = CHANNELS[j]
            nb = cj // 16
            shift = nb.bit_length() - 1

            @plsc.parallel_loop(0, CHUNK * nb)
            def _blk_loop(i, _rows=rows[j], _coff=CH_OFF[j]):
                p = lax.shift_right_logical(i, shift)
                off = lax.shift_left(i & (nb - 1), 4)
                x0 = _rows[p, pl.ds(off, 16)]
                x1 = _rows[CHUNK + p, pl.ds(off, 16)]
                x2 = _rows[2 * CHUNK + p, pl.ds(off, 16)]
                mx = lax.max(lax.max(x0, x1), x2)
                mn = (x0 + x1 + x2) * third
                d0 = x0 - mn
                d1 = x1 - mn
                d2 = x2 - mn
                var = (d0 * d0 + d1 * d1 + d2 * d2) * third
                varc = lax.max(var, jnp.float32(1e-35))
                iy = jnp.int32(0x5F3759DF) - lax.shift_right_logical(
                    lax.bitcast_convert_type(varc, jnp.int32), 1)
                y = lax.bitcast_convert_type(iy, jnp.float32)
                hv = varc * jnp.float32(0.5)
                y = y * (jnp.float32(1.5) - hv * y * y)
                y = y * (jnp.float32(1.5) - hv * y * y)
                y = y * (jnp.float32(1.5) - hv * y * y)
                sd = varc * y
                col = p * OUT_COLS + 3 + _coff + off
                outbuf[pl.ds(col, 16)] = mx
                outbuf[pl.ds(col + C_TOTAL, 16)] = mn
                outbuf[pl.ds(col + 2 * C_TOTAL, 16)] = sd
        pltpu.sync_copy(outbuf, out.at[pl.ds(c * OUT_CHUNK, OUT_CHUNK)])
        return carry

    lax.fori_loop(0, count, chunk_body, 0)


def kernel(inputs, img_feat_0, img_feat_1, img_feat_2, img_feat_3, cameras):
    idxp = _flat_indices(inputs, cameras)
    coords = jnp.pad(inputs.reshape(N_CHUNKS, CHUNK, 3),
                     ((0, 0), (0, 0), (0, 13)))
    feats = (img_feat_0, img_feat_1, img_feat_2, img_feat_3)
    tables = [f.reshape(N_VIEWS * s * s, c)
              for f, s, c in zip(feats, SCALES, CHANNELS)]

    mesh = plsc.VectorSubcoreMesh(core_axis_name="c", subcore_axis_name="s")
    run = functools.partial(
        pl.kernel,
        out_type=jax.ShapeDtypeStruct((N_POINTS * OUT_COLS,), jnp.float32),
        mesh=mesh,
        compiler_params=pltpu.CompilerParams(use_tc_tiling_on_sc=False),
        scratch_types=[
            pltpu.VMEM((4, N_VIEWS * CHUNK), jnp.int32),
            pltpu.VMEM((CHUNK, 16), jnp.float32),
            pltpu.VMEM((N_VIEWS * CHUNK, CHANNELS[0]), jnp.float32),
            pltpu.VMEM((N_VIEWS * CHUNK, CHANNELS[1]), jnp.float32),
            pltpu.VMEM((N_VIEWS * CHUNK, CHANNELS[2]), jnp.float32),
            pltpu.VMEM((N_VIEWS * CHUNK, CHANNELS[3]), jnp.float32),
            pltpu.VMEM((OUT_CHUNK,), jnp.float32),
            pltpu.SemaphoreType.DMA,
            pltpu.SemaphoreType.DMA,
            pltpu.SemaphoreType.DMA,
            pltpu.SemaphoreType.DMA,
        ],
    )(_sc_body)
    out = run(tables[0], tables[1], tables[2], tables[3], idxp, coords)
    return out.reshape(N_POINTS, OUT_COLS)


# speedup vs baseline: 1.9907x; 1.0609x over previous
import functools

import jax
import jax.numpy as jnp
import numpy as np
from jax import lax
from jax.experimental import pallas as pl
from jax.experimental.pallas import tpu as pltpu
from jax.experimental.pallas import tpu_sc as plsc

N_POINTS = 10000
N_VIEWS = 3
SCALES = (56, 28, 14, 7)
CHANNELS = (64, 128, 256, 512)
C_TOTAL = 960
OUT_COLS = 3 + 3 * C_TOTAL
CHUNK = 8
N_CHUNKS = N_POINTS // CHUNK
G_ROWS = N_VIEWS * CHUNK
OUT_CHUNK = CHUNK * OUT_COLS

NUM_CORES = 2
NUM_SUBCORES = 16
NUM_WORKERS = NUM_CORES * NUM_SUBCORES
BASE_CHUNKS = N_CHUNKS // NUM_WORKERS
EXTRA = N_CHUNKS - BASE_CHUNKS * NUM_WORKERS
MAX_CHUNKS = BASE_CHUNKS + 1
PAD_CHUNKS = NUM_WORKERS * MAX_CHUNKS

CH_OFF = (0, 64, 192, 448)


def _normal(v):
    return v / jnp.linalg.norm(v)


def _camera_mat(param):
    theta = param[0] * np.pi / 180.0
    camy = param[3] * jnp.sin(param[1] * np.pi / 180.0)
    lens = param[3] * jnp.cos(param[1] * np.pi / 180.0)
    camx = lens * jnp.cos(theta)
    camz = lens * jnp.sin(theta)
    Z = jnp.stack([camx, camy, camz])
    x = camy * jnp.cos(theta + np.pi)
    z = camy * jnp.sin(theta + np.pi)
    Y = jnp.stack([x, lens, z])
    X = jnp.cross(Y, Z)
    cm_mat = jnp.stack([_normal(X), _normal(Y), _normal(Z)])
    return cm_mat, Z


def _camera_trans(param, xyz):
    c, o = _camera_mat(param)
    return (xyz - o) @ c.T


def _camera_trans_inv(param, xyz):
    c, o = _camera_mat(param)
    return xyz @ jnp.linalg.inv(c.T) + o


def _flat_indices(inputs, cameras):
    flat = [[] for _ in SCALES]
    for i in range(N_VIEWS):
        point_origin = _camera_trans_inv(cameras[0], inputs)
        point_current = _camera_trans(cameras[i], point_origin)
        X = point_current[:, 0]
        Y = point_current[:, 1]
        Z = point_current[:, 2]
        h = 248.0 * ((-Y) / (-Z)) + 112.0
        w = 248.0 * (X / (-Z)) + 112.0
        h = jnp.minimum(jnp.maximum(h, 0.0), 223.0)
        w = jnp.minimum(jnp.maximum(w, 0.0), 223.0)
        n = jnp.full(h.shape, float(i), dtype=jnp.float32)
        indeces = jnp.stack([n, h, w], 1)
        for j, s in enumerate(SCALES):
            idx = (indeces / (224.0 / float(s))).astype(jnp.int32)
            flat[j].append((idx[:, 0] * s + idx[:, 1]) * s + idx[:, 2])
    packed = []
    for j in range(len(SCALES)):
        a = jnp.stack(flat[j], 0)
        a = a.reshape(N_VIEWS, N_CHUNKS, CHUNK).transpose(1, 0, 2)
        packed.append(a.reshape(N_CHUNKS, G_ROWS))
    out = jnp.stack(packed, 1)
    return jnp.pad(out, ((0, PAD_CHUNKS - N_CHUNKS), (0, 0), (0, 0)))


def _sc_body(t0, t1, t2, t3, idxp, coords, out,
             idxall, coordall, r0, r1, r2, r3, outbuf,
             sg00, sg01, sg02, sg03, sg10, sg11, sg12, sg13, so0, so1):
    tables = (t0, t1, t2, t3)
    rows = (r0, r1, r2, r3)
    sg = ((sg00, sg01, sg02, sg03), (sg10, sg11, sg12, sg13))
    so = (so0, so1)
    wid = lax.axis_index("s") * NUM_CORES + lax.axis_index("c")
    start = wid * BASE_CHUNKS + lax.min(wid, EXTRA)
    count = BASE_CHUNKS + jnp.where(wid < EXTRA, 1, 0)

    pltpu.sync_copy(idxp.at[pl.ds(start, MAX_CHUNKS)], idxall)
    pltpu.sync_copy(coords.at[pl.ds(start, MAX_CHUNKS)], coordall)

    third = jnp.float32(1.0 / 3.0)

    def gather_descs(t, s):
        return [pltpu.make_async_copy(
            tables[j].at[idxall.at[t, j]], rows[j].at[s], sg[s][j])
            for j in range(4)]

    def issue_gathers(t, s):
        for j in range(4):
            pltpu.async_copy(
                tables[j].at[idxall.at[t, j]], rows[j].at[s], sg[s][j])

    def out_desc(t, s):
        return pltpu.make_async_copy(
            outbuf.at[s], out.at[pl.ds((start + t) * CHUNK, CHUNK), :], so[s])

    issue_gathers(0, 0)

    def half_body(t, s):
        s1 = 1 - s

        @pl.when(t + 1 < count)
        def _prefetch():
            issue_gathers(t + 1, s1)

        for d in gather_descs(t, s):
            d.wait()

        @pl.when(t >= 2)
        def _drain_out():
            out_desc(t - 2, s).wait()

        @plsc.parallel_loop(0, CHUNK)
        def _coord_loop(p):
            outbuf[s, p, pl.ds(0, 16)] = coordall[t, p, :]

        for j in range(4):
            cj = CHANNELS[j]
            nb = cj // 16
            shift = nb.bit_length() - 1

            @plsc.parallel_loop(0, CHUNK * nb)
            def _blk_loop(i, _rows=rows[j], _coff=CH_OFF[j]):
                p = lax.shift_right_logical(i, shift)
                off = lax.shift_left(i & (nb - 1), 4)
                x0 = _rows[s, p, pl.ds(off, 16)]
                x1 = _rows[s, CHUNK + p, pl.ds(off, 16)]
                x2 = _rows[s, 2 * CHUNK + p, pl.ds(off, 16)]
                mx = lax.max(lax.max(x0, x1), x2)
                mn = (x0 + x1 + x2) * third
                d0 = x0 - mn
                d1 = x1 - mn
                d2 = x2 - mn
                var = (d0 * d0 + d1 * d1 + d2 * d2) * third
                varc = lax.max(var, jnp.float32(1e-35))
                iy = jnp.int32(0x5F3759DF) - lax.shift_right_logical(
                    lax.bitcast_convert_type(varc, jnp.int32), 1)
                y = lax.bitcast_convert_type(iy, jnp.float32)
                hv = varc * jnp.float32(0.5)
                y = y * (jnp.float32(1.5) - hv * y * y)
                y = y * (jnp.float32(1.5) - hv * y * y)
                y = y * (jnp.float32(1.5) - hv * y * y)
                sd = varc * y
                col = 3 + _coff + off
                outbuf[s, p, pl.ds(col, 16)] = mx
                outbuf[s, p, pl.ds(col + C_TOTAL, 16)] = mn
                outbuf[s, p, pl.ds(col + 2 * C_TOTAL, 16)] = sd

        pltpu.async_copy(
            outbuf.at[s], out.at[pl.ds((start + t) * CHUNK, CHUNK), :], so[s])

    def pair_body(t2, carry):
        for par in (0, 1):
            t = 2 * t2 + par

            @pl.when(t < count)
            def _half(t=t, par=par):
                half_body(t, par)

        return carry

    lax.fori_loop(0, (count + 1) // 2, pair_body, 0)

    out_desc(0, 0).wait()
    out_desc(0, 1).wait()


def kernel(inputs, img_feat_0, img_feat_1, img_feat_2, img_feat_3, cameras):
    idxp = _flat_indices(inputs, cameras)
    coords = jnp.pad(inputs.reshape(N_CHUNKS, CHUNK, 3),
                     ((0, PAD_CHUNKS - N_CHUNKS), (0, 0), (0, 13)))
    feats = (img_feat_0, img_feat_1, img_feat_2, img_feat_3)
    tables = [f.reshape(N_VIEWS * s * s, c)
              for f, s, c in zip(feats, SCALES, CHANNELS)]

    mesh = plsc.VectorSubcoreMesh(core_axis_name="c", subcore_axis_name="s")
    run = functools.partial(
        pl.kernel,
        out_type=jax.ShapeDtypeStruct((N_POINTS, OUT_COLS), jnp.float32),
        mesh=mesh,
        compiler_params=pltpu.CompilerParams(use_tc_tiling_on_sc=False),
        scratch_types=[
            pltpu.VMEM((MAX_CHUNKS, 4, G_ROWS), jnp.int32),
            pltpu.VMEM((MAX_CHUNKS, CHUNK, 16), jnp.float32),
            pltpu.VMEM((2, G_ROWS, CHANNELS[0]), jnp.float32),
            pltpu.VMEM((2, G_ROWS, CHANNELS[1]), jnp.float32),
            pltpu.VMEM((2, G_ROWS, CHANNELS[2]), jnp.float32),
            pltpu.VMEM((2, G_ROWS, CHANNELS[3]), jnp.float32),
            pltpu.VMEM((2, CHUNK, OUT_COLS), jnp.float32),
        ] + [pltpu.SemaphoreType.DMA] * 10,
    )(_sc_body)
    return run(tables[0], tables[1], tables[2], tables[3], idxp, coords)


# speedup vs baseline: 2.3631x; 1.1871x over previous
import functools

import jax
import jax.numpy as jnp
import numpy as np
from jax import lax
from jax.experimental import pallas as pl
from jax.experimental.pallas import tpu as pltpu
from jax.experimental.pallas import tpu_sc as plsc

N_POINTS = 10000
N_VIEWS = 3
SCALES = (56, 28, 14, 7)
CHANNELS = (64, 128, 256, 512)
C_TOTAL = 960
OUT_COLS = 3 + 3 * C_TOTAL
CHUNK = 16
N_CHUNKS = N_POINTS // CHUNK
G_ROWS = N_VIEWS * CHUNK

NUM_CORES = 2
NUM_SUBCORES = 16
NUM_WORKERS = NUM_CORES * NUM_SUBCORES
BASE_CHUNKS = N_CHUNKS // NUM_WORKERS
EXTRA = N_CHUNKS - BASE_CHUNKS * NUM_WORKERS
MAX_CHUNKS = BASE_CHUNKS + 1
PAD_CHUNKS = NUM_WORKERS * MAX_CHUNKS

CH_OFF = (0, 64, 192, 448)


def _normal(v):
    return v / jnp.linalg.norm(v)


def _camera_mat(param):
    theta = param[0] * np.pi / 180.0
    camy = param[3] * jnp.sin(param[1] * np.pi / 180.0)
    lens = param[3] * jnp.cos(param[1] * np.pi / 180.0)
    camx = lens * jnp.cos(theta)
    camz = lens * jnp.sin(theta)
    Z = jnp.stack([camx, camy, camz])
    x = camy * jnp.cos(theta + np.pi)
    z = camy * jnp.sin(theta + np.pi)
    Y = jnp.stack([x, lens, z])
    X = jnp.cross(Y, Z)
    cm_mat = jnp.stack([_normal(X), _normal(Y), _normal(Z)])
    return cm_mat, Z


def _camera_trans(param, xyz):
    c, o = _camera_mat(param)
    return (xyz - o) @ c.T


def _camera_trans_inv(param, xyz):
    c, o = _camera_mat(param)
    return xyz @ jnp.linalg.inv(c.T) + o


def _flat_indices(inputs, cameras):
    flat = [[] for _ in SCALES]
    for i in range(N_VIEWS):
        point_origin = _camera_trans_inv(cameras[0], inputs)
        point_current = _camera_trans(cameras[i], point_origin)
        X = point_current[:, 0]
        Y = point_current[:, 1]
        Z = point_current[:, 2]
        h = 248.0 * ((-Y) / (-Z)) + 112.0
        w = 248.0 * (X / (-Z)) + 112.0
        h = jnp.minimum(jnp.maximum(h, 0.0), 223.0)
        w = jnp.minimum(jnp.maximum(w, 0.0), 223.0)
        n = jnp.full(h.shape, float(i), dtype=jnp.float32)
        indeces = jnp.stack([n, h, w], 1)
        for j, s in enumerate(SCALES):
            idx = (indeces / (224.0 / float(s))).astype(jnp.int32)
            flat[j].append((idx[:, 0] * s + idx[:, 1]) * s + idx[:, 2])
    packed = []
    for j in range(len(SCALES)):
        a = jnp.stack(flat[j], 0)
        a = a.reshape(N_VIEWS, N_CHUNKS, CHUNK).transpose(1, 0, 2)
        packed.append(a.reshape(N_CHUNKS, G_ROWS))
    out = jnp.stack(packed, 1)
    return jnp.pad(out, ((0, PAD_CHUNKS - N_CHUNKS), (0, 0), (0, 0)))


def _sc_body(t0, t1, t2, t3, idxp, coords, out,
             idxall, coordall, r0, r1, r2, r3, outbuf,
             sg0, sg1, sg2, sg3, so):
    tables = (t0, t1, t2, t3)
    rows = (r0, r1, r2, r3)
    sg = (sg0, sg1, sg2, sg3)
    wid = lax.axis_index("s") * NUM_CORES + lax.axis_index("c")
    start = wid * BASE_CHUNKS + lax.min(wid, EXTRA)
    count = BASE_CHUNKS + jnp.where(wid < EXTRA, 1, 0)

    pltpu.sync_copy(idxp.at[pl.ds(start, MAX_CHUNKS)], idxall)
    pltpu.sync_copy(coords.at[pl.ds(start, MAX_CHUNKS)], coordall)

    third = jnp.float32(1.0 / 3.0)
    lanes = lax.broadcasted_iota(jnp.int32, (CHUNK,), 0)

    def issue_gathers(t):
        for j in range(4):
            pltpu.async_copy(
                tables[j].at[idxall.at[t, j]], rows[j], sg[j])

    def wait_gathers(t):
        for j in range(4):
            pltpu.make_async_copy(
                tables[j].at[idxall.at[t, j]], rows[j], sg[j]).wait()

    def out_desc(t):
        return pltpu.make_async_copy(
            outbuf, out.at[:, pl.ds((start + t) * CHUNK, CHUNK)], so)

    def chunk_body(t, carry):
        issue_gathers(t)

        @pl.when(t >= 1)
        def _drain_out():
            out_desc(t - 1).wait()

        wait_gathers(t)

        for d in range(3):
            outbuf[d, pl.ds(0, CHUNK)] = coordall[t, d, :]

        for j in range(4):
            cj = CHANNELS[j]

            @plsc.parallel_loop(0, cj)
            def _ch_loop(c, _rows=rows[j], _coff=CH_OFF[j]):
                cvec = jnp.full((CHUNK,), 0, jnp.int32) + c
                x0 = plsc.load_gather(_rows, [lanes, cvec])
                x1 = plsc.load_gather(_rows, [lanes + CHUNK, cvec])
                x2 = plsc.load_gather(_rows, [lanes + 2 * CHUNK, cvec])
                mx = lax.max(lax.max(x0, x1), x2)
                mn = (x0 + x1 + x2) * third
                d0 = x0 - mn
                d1 = x1 - mn
                d2 = x2 - mn
                var = (d0 * d0 + d1 * d1 + d2 * d2) * third
                varc = lax.max(var, jnp.float32(1e-35))
                iy = jnp.int32(0x5F3759DF) - lax.shift_right_logical(
                    lax.bitcast_convert_type(varc, jnp.int32), 1)
                y = lax.bitcast_convert_type(iy, jnp.float32)
                hv = varc * jnp.float32(0.5)
                y = y * (jnp.float32(1.5) - hv * y * y)
                y = y * (jnp.float32(1.5) - hv * y * y)
                y = y * (jnp.float32(1.5) - hv * y * y)
                sd = varc * y
                row = 3 + _coff + c
                outbuf[row, pl.ds(0, CHUNK)] = mx
                outbuf[row + C_TOTAL, pl.ds(0, CHUNK)] = mn
                outbuf[row + 2 * C_TOTAL, pl.ds(0, CHUNK)] = sd

        pltpu.async_copy(
            outbuf, out.at[:, pl.ds((start + t) * CHUNK, CHUNK)], so)
        return carry

    lax.fori_loop(0, count, chunk_body, 0)
    out_desc(count - 1).wait()


def kernel(inputs, img_feat_0, img_feat_1, img_feat_2, img_feat_3, cameras):
    idxp = _flat_indices(inputs, cameras)
    coords = jnp.pad(inputs.reshape(N_CHUNKS, CHUNK, 3),
                     ((0, PAD_CHUNKS - N_CHUNKS), (0, 0), (0, 0)))
    coords = coords.transpose(0, 2, 1)
    feats = (img_feat_0, img_feat_1, img_feat_2, img_feat_3)
    tables = [f.reshape(N_VIEWS * s * s, c)
              for f, s, c in zip(feats, SCALES, CHANNELS)]

    mesh = plsc.VectorSubcoreMesh(core_axis_name="c", subcore_axis_name="s")
    run = functools.partial(
        pl.kernel,
        out_type=jax.ShapeDtypeStruct((OUT_COLS, N_POINTS), jnp.float32),
        mesh=mesh,
        compiler_params=pltpu.CompilerParams(use_tc_tiling_on_sc=False,
                                             needs_layout_passes=False),
        scratch_types=[
            pltpu.VMEM((MAX_CHUNKS, 4, G_ROWS), jnp.int32),
            pltpu.VMEM((MAX_CHUNKS, 3, CHUNK), jnp.float32),
            pltpu.VMEM((G_ROWS, CHANNELS[0]), jnp.float32),
            pltpu.VMEM((G_ROWS, CHANNELS[1]), jnp.float32),
            pltpu.VMEM((G_ROWS, CHANNELS[2]), jnp.float32),
            pltpu.VMEM((G_ROWS, CHANNELS[3]), jnp.float32),
            pltpu.VMEM((OUT_COLS, CHUNK), jnp.float32),
        ] + [pltpu.SemaphoreType.DMA] * 5,
    )(_sc_body)
    out = run(tables[0], tables[1], tables[2], tables[3], idxp, coords)
    return out.T


# speedup vs baseline: 2.4917x; 1.0544x over previous
import functools

import jax
import jax.numpy as jnp
import numpy as np
from jax import lax
from jax.experimental import pallas as pl
from jax.experimental.pallas import tpu as pltpu
from jax.experimental.pallas import tpu_sc as plsc

N_POINTS = 10000
N_VIEWS = 3
SCALES = (56, 28, 14, 7)
CHANNELS = (64, 128, 256, 512)
C_TOTAL = 960
OUT_COLS = 3 + 3 * C_TOTAL
CHUNK = 16
N_CHUNKS = N_POINTS // CHUNK
G_ROWS = N_VIEWS * CHUNK

NUM_CORES = 2
NUM_SUBCORES = 16
NUM_WORKERS = NUM_CORES * NUM_SUBCORES
BASE_CHUNKS = N_CHUNKS // NUM_WORKERS
EXTRA = N_CHUNKS - BASE_CHUNKS * NUM_WORKERS
MAX_CHUNKS = BASE_CHUNKS + 1
PAD_CHUNKS = NUM_WORKERS * MAX_CHUNKS

CH_OFF = (0, 64, 192, 448)


def _normal(v):
    return v / jnp.linalg.norm(v)


def _camera_mat(param):
    theta = param[0] * np.pi / 180.0
    camy = param[3] * jnp.sin(param[1] * np.pi / 180.0)
    lens = param[3] * jnp.cos(param[1] * np.pi / 180.0)
    camx = lens * jnp.cos(theta)
    camz = lens * jnp.sin(theta)
    Z = jnp.stack([camx, camy, camz])
    x = camy * jnp.cos(theta + np.pi)
    z = camy * jnp.sin(theta + np.pi)
    Y = jnp.stack([x, lens, z])
    X = jnp.cross(Y, Z)
    cm_mat = jnp.stack([_normal(X), _normal(Y), _normal(Z)])
    return cm_mat, Z


def _camera_trans(param, xyz):
    c, o = _camera_mat(param)
    return (xyz - o) @ c.T


def _camera_trans_inv(param, xyz):
    c, o = _camera_mat(param)
    return xyz @ jnp.linalg.inv(c.T) + o


def _flat_indices(inputs, cameras):
    flat = [[] for _ in SCALES]
    for i in range(N_VIEWS):
        point_origin = _camera_trans_inv(cameras[0], inputs)
        point_current = _camera_trans(cameras[i], point_origin)
        X = point_current[:, 0]
        Y = point_current[:, 1]
        Z = point_current[:, 2]
        h = 248.0 * ((-Y) / (-Z)) + 112.0
        w = 248.0 * (X / (-Z)) + 112.0
        h = jnp.minimum(jnp.maximum(h, 0.0), 223.0)
        w = jnp.minimum(jnp.maximum(w, 0.0), 223.0)
        n = jnp.full(h.shape, float(i), dtype=jnp.float32)
        indeces = jnp.stack([n, h, w], 1)
        for j, s in enumerate(SCALES):
            idx = (indeces / (224.0 / float(s))).astype(jnp.int32)
            flat[j].append((idx[:, 0] * s + idx[:, 1]) * s + idx[:, 2])
    packed = []
    for j in range(len(SCALES)):
        a = jnp.stack(flat[j], 0)
        a = a.reshape(N_VIEWS, N_CHUNKS, CHUNK).transpose(1, 0, 2)
        packed.append(a.reshape(N_CHUNKS, G_ROWS))
    out = jnp.stack(packed, 1)
    return jnp.pad(out, ((0, PAD_CHUNKS - N_CHUNKS), (0, 0), (0, 0)))


def _sc_body(t0, t1, t2, t3, idxp, coords, out,
             idxall, coordall, r0, r1, r2, r3, outbuf,
             sg0, sg1, sg2, sg3, so):
    tables = (t0, t1, t2, t3)
    rows = (r0, r1, r2, r3)
    sg = (sg0, sg1, sg2, sg3)
    wid = lax.axis_index("s") * NUM_CORES + lax.axis_index("c")
    start = wid * BASE_CHUNKS + lax.min(wid, EXTRA)
    count = BASE_CHUNKS + jnp.where(wid < EXTRA, 1, 0)

    pltpu.sync_copy(idxp.at[pl.ds(start, MAX_CHUNKS)], idxall)
    pltpu.sync_copy(coords.at[pl.ds(start, MAX_CHUNKS)], coordall)

    third = jnp.float32(1.0 / 3.0)
    lanes = lax.broadcasted_iota(jnp.int32, (CHUNK,), 0)

    def issue_gathers(t):
        for j in range(4):
            pltpu.async_copy(
                tables[j].at[idxall.at[t, j]], rows[j], sg[j])

    def wait_gathers(t):
        for j in range(4):
            pltpu.make_async_copy(
                tables[j].at[idxall.at[t, j]], rows[j], sg[j]).wait()

    def out_desc(t):
        return pltpu.make_async_copy(
            outbuf, out.at[:, pl.ds((start + t) * CHUNK, CHUNK)], so)

    def chunk_body(t, carry):
        issue_gathers(t)

        @pl.when(t >= 1)
        def _drain_out():
            out_desc(t - 1).wait()

        wait_gathers(t)

        for d in range(3):
            outbuf[d, pl.ds(0, CHUNK)] = coordall[t, d, :]

        for j in range(4):
            cj = CHANNELS[j]

            @plsc.parallel_loop(0, cj, unroll=4)
            def _ch_loop(c, _rows=rows[j], _coff=CH_OFF[j]):
                cvec = jnp.full((CHUNK,), 0, jnp.int32) + c
                x0 = plsc.load_gather(_rows, [lanes, cvec])
                x1 = plsc.load_gather(_rows, [lanes + CHUNK, cvec])
                x2 = plsc.load_gather(_rows, [lanes + 2 * CHUNK, cvec])
                mx = lax.max(lax.max(x0, x1), x2)
                mn = (x0 + x1 + x2) * third
                d0 = x0 - mn
                d1 = x1 - mn
                d2 = x2 - mn
                var = (d0 * d0 + d1 * d1 + d2 * d2) * third
                varc = lax.max(var, jnp.float32(1e-35))
                iy = jnp.int32(0x5F3759DF) - lax.shift_right_logical(
                    lax.bitcast_convert_type(varc, jnp.int32), 1)
                y = lax.bitcast_convert_type(iy, jnp.float32)
                hv = varc * jnp.float32(0.5)
                y = y * (jnp.float32(1.5) - hv * y * y)
                y = y * (jnp.float32(1.5) - hv * y * y)
                y = y * (jnp.float32(1.5) - hv * y * y)
                sd = varc * y
                row = 3 + _coff + c
                outbuf[row, pl.ds(0, CHUNK)] = mx
                outbuf[row + C_TOTAL, pl.ds(0, CHUNK)] = mn
                outbuf[row + 2 * C_TOTAL, pl.ds(0, CHUNK)] = sd

        pltpu.async_copy(
            outbuf, out.at[:, pl.ds((start + t) * CHUNK, CHUNK)], so)
        return carry

    lax.fori_loop(0, count, chunk_body, 0)
    out_desc(count - 1).wait()


def kernel(inputs, img_feat_0, img_feat_1, img_feat_2, img_feat_3, cameras):
    idxp = _flat_indices(inputs, cameras)
    coords = jnp.pad(inputs.reshape(N_CHUNKS, CHUNK, 3),
                     ((0, PAD_CHUNKS - N_CHUNKS), (0, 0), (0, 0)))
    coords = coords.transpose(0, 2, 1)
    feats = (img_feat_0, img_feat_1, img_feat_2, img_feat_3)
    tables = [f.reshape(N_VIEWS * s * s, c)
              for f, s, c in zip(feats, SCALES, CHANNELS)]

    mesh = plsc.VectorSubcoreMesh(core_axis_name="c", subcore_axis_name="s")
    run = functools.partial(
        pl.kernel,
        out_type=jax.ShapeDtypeStruct((OUT_COLS, N_POINTS), jnp.float32),
        mesh=mesh,
        compiler_params=pltpu.CompilerParams(use_tc_tiling_on_sc=False,
                                             needs_layout_passes=False),
        scratch_types=[
            pltpu.VMEM((MAX_CHUNKS, 4, G_ROWS), jnp.int32),
            pltpu.VMEM((MAX_CHUNKS, 3, CHUNK), jnp.float32),
            pltpu.VMEM((G_ROWS, CHANNELS[0]), jnp.float32),
            pltpu.VMEM((G_ROWS, CHANNELS[1]), jnp.float32),
            pltpu.VMEM((G_ROWS, CHANNELS[2]), jnp.float32),
            pltpu.VMEM((G_ROWS, CHANNELS[3]), jnp.float32),
            pltpu.VMEM((OUT_COLS, CHUNK), jnp.float32),
        ] + [pltpu.SemaphoreType.DMA] * 5,
    )(_sc_body)
    out = run(tables[0], tables[1], tables[2], tables[3], idxp, coords)
    return out.T


# speedup vs baseline: 2.5074x; 1.0063x over previous
import functools

import jax
import jax.numpy as jnp
import numpy as np
from jax import lax
from jax.experimental import pallas as pl
from jax.experimental.pallas import tpu as pltpu
from jax.experimental.pallas import tpu_sc as plsc

N_POINTS = 10000
N_VIEWS = 3
SCALES = (56, 28, 14, 7)
CHANNELS = (64, 128, 256, 512)
C_TOTAL = 960
OUT_COLS = 3 + 3 * C_TOTAL
CHUNK = 16
N_CHUNKS = N_POINTS // CHUNK
G_ROWS = N_VIEWS * CHUNK

NUM_CORES = 2
NUM_SUBCORES = 16
NUM_WORKERS = NUM_CORES * NUM_SUBCORES
BASE_CHUNKS = N_CHUNKS // NUM_WORKERS
EXTRA = N_CHUNKS - BASE_CHUNKS * NUM_WORKERS
MAX_CHUNKS = BASE_CHUNKS + 1
PAD_CHUNKS = NUM_WORKERS * MAX_CHUNKS

CH_OFF = (0, 64, 192, 448)


def _normal(v):
    return v / jnp.linalg.norm(v)


def _camera_mat(param):
    theta = param[0] * np.pi / 180.0
    camy = param[3] * jnp.sin(param[1] * np.pi / 180.0)
    lens = param[3] * jnp.cos(param[1] * np.pi / 180.0)
    camx = lens * jnp.cos(theta)
    camz = lens * jnp.sin(theta)
    Z = jnp.stack([camx, camy, camz])
    x = camy * jnp.cos(theta + np.pi)
    z = camy * jnp.sin(theta + np.pi)
    Y = jnp.stack([x, lens, z])
    X = jnp.cross(Y, Z)
    cm_mat = jnp.stack([_normal(X), _normal(Y), _normal(Z)])
    return cm_mat, Z


def _camera_trans(param, xyz):
    c, o = _camera_mat(param)
    return (xyz - o) @ c.T


def _camera_trans_inv(param, xyz):
    c, o = _camera_mat(param)
    return xyz @ jnp.linalg.inv(c.T) + o


def _flat_indices(inputs, cameras):
    flat = [[] for _ in SCALES]
    for i in range(N_VIEWS):
        point_origin = _camera_trans_inv(cameras[0], inputs)
        point_current = _camera_trans(cameras[i], point_origin)
        X = point_current[:, 0]
        Y = point_current[:, 1]
        Z = point_current[:, 2]
        h = 248.0 * ((-Y) / (-Z)) + 112.0
        w = 248.0 * (X / (-Z)) + 112.0
        h = jnp.minimum(jnp.maximum(h, 0.0), 223.0)
        w = jnp.minimum(jnp.maximum(w, 0.0), 223.0)
        n = jnp.full(h.shape, float(i), dtype=jnp.float32)
        indeces = jnp.stack([n, h, w], 1)
        for j, s in enumerate(SCALES):
            idx = (indeces / (224.0 / float(s))).astype(jnp.int32)
            flat[j].append((idx[:, 0] * s + idx[:, 1]) * s + idx[:, 2])
    packed = []
    for j in range(len(SCALES)):
        a = jnp.stack(flat[j], 0)
        a = a.reshape(N_VIEWS, N_CHUNKS, CHUNK).transpose(1, 0, 2)
        packed.append(a.reshape(N_CHUNKS, G_ROWS))
    out = jnp.stack(packed, 1)
    return jnp.pad(out, ((0, PAD_CHUNKS - N_CHUNKS), (0, 0), (0, 0)))


def _sc_body(t0, t1, t2, t3, idxp, coords, out,
             idxall, coordall, r0, r1, r2, r3, outbuf,
             sg0, sg1, sg2, sg3, so):
    tables = (t0, t1, t2, t3)
    rows = (r0, r1, r2, r3)
    sg = (sg0, sg1, sg2, sg3)
    wid = lax.axis_index("s") * NUM_CORES + lax.axis_index("c")
    start = wid * BASE_CHUNKS + lax.min(wid, EXTRA)
    count = BASE_CHUNKS + jnp.where(wid < EXTRA, 1, 0)

    pltpu.sync_copy(idxp.at[pl.ds(start, MAX_CHUNKS)], idxall)
    pltpu.sync_copy(coords.at[pl.ds(start, MAX_CHUNKS)], coordall)

    third = jnp.float32(1.0 / 3.0)
    lanes = lax.broadcasted_iota(jnp.int32, (CHUNK,), 0)

    def issue_gathers(t):
        for j in range(4):
            pltpu.async_copy(
                tables[j].at[idxall.at[t, j]], rows[j], sg[j])

    def wait_gathers(t):
        for j in range(4):
            pltpu.make_async_copy(
                tables[j].at[idxall.at[t, j]], rows[j], sg[j]).wait()

    def out_desc(t):
        return pltpu.make_async_copy(
            outbuf, out.at[:, pl.ds((start + t) * CHUNK, CHUNK)], so)

    def chunk_body(t, carry):
        issue_gathers(t)

        @pl.when(t >= 1)
        def _drain_out():
            out_desc(t - 1).wait()

        wait_gathers(t)

        for d in range(3):
            outbuf[d, pl.ds(0, CHUNK)] = coordall[t, d, :]

        for j in range(4):
            cj = CHANNELS[j]

            @plsc.parallel_loop(0, cj, unroll=8)
            def _ch_loop(c, _rows=rows[j], _coff=CH_OFF[j]):
                cvec = jnp.full((CHUNK,), 0, jnp.int32) + c
                x0 = plsc.load_gather(_rows, [lanes, cvec])
                x1 = plsc.load_gather(_rows, [lanes + CHUNK, cvec])
                x2 = plsc.load_gather(_rows, [lanes + 2 * CHUNK, cvec])
                mx = lax.max(lax.max(x0, x1), x2)
                mn = (x0 + x1 + x2) * third
                d0 = x0 - mn
                d1 = x1 - mn
                d2 = x2 - mn
                var = (d0 * d0 + d1 * d1 + d2 * d2) * third
                varc = lax.max(var, jnp.float32(1e-35))
                iy = jnp.int32(0x5F3759DF) - lax.shift_right_logical(
                    lax.bitcast_convert_type(varc, jnp.int32), 1)
                y = lax.bitcast_convert_type(iy, jnp.float32)
                hv = varc * jnp.float32(0.5)
                y = y * (jnp.float32(1.5) - hv * y * y)
                y = y * (jnp.float32(1.5) - hv * y * y)
                sd = varc * y
                row = 3 + _coff + c
                outbuf[row, pl.ds(0, CHUNK)] = mx
                outbuf[row + C_TOTAL, pl.ds(0, CHUNK)] = mn
                outbuf[row + 2 * C_TOTAL, pl.ds(0, CHUNK)] = sd

        pltpu.async_copy(
            outbuf, out.at[:, pl.ds((start + t) * CHUNK, CHUNK)], so)
        return carry

    lax.fori_loop(0, count, chunk_body, 0)
    out_desc(count - 1).wait()


def kernel(inputs, img_feat_0, img_feat_1, img_feat_2, img_feat_3, cameras):
    idxp = _flat_indices(inputs, cameras)
    coords = jnp.pad(inputs.reshape(N_CHUNKS, CHUNK, 3),
                     ((0, PAD_CHUNKS - N_CHUNKS), (0, 0), (0, 0)))
    coords = coords.transpose(0, 2, 1)
    feats = (img_feat_0, img_feat_1, img_feat_2, img_feat_3)
    tables = [f.reshape(N_VIEWS * s * s, c)
              for f, s, c in zip(feats, SCALES, CHANNELS)]

    mesh = plsc.VectorSubcoreMesh(core_axis_name="c", subcore_axis_name="s")
    run = functools.partial(
        pl.kernel,
        out_type=jax.ShapeDtypeStruct((OUT_COLS, N_POINTS), jnp.float32),
        mesh=mesh,
        compiler_params=pltpu.CompilerParams(use_tc_tiling_on_sc=False,
                                             needs_layout_passes=False),
        scratch_types=[
            pltpu.VMEM((MAX_CHUNKS, 4, G_ROWS), jnp.int32),
            pltpu.VMEM((MAX_CHUNKS, 3, CHUNK), jnp.float32),
            pltpu.VMEM((G_ROWS, CHANNELS[0]), jnp.float32),
            pltpu.VMEM((G_ROWS, CHANNELS[1]), jnp.float32),
            pltpu.VMEM((G_ROWS, CHANNELS[2]), jnp.float32),
            pltpu.VMEM((G_ROWS, CHANNELS[3]), jnp.float32),
            pltpu.VMEM((OUT_COLS, CHUNK), jnp.float32),
        ] + [pltpu.SemaphoreType.DMA] * 5,
    )(_sc_body)
    out = run(tables[0], tables[1], tables[2], tables[3], idxp, coords)
    return out.T
